# baseline (speedup 1.0000x reference)
import jax
import jax.numpy as jnp
from jax.experimental import pallas as pl

N = 10000
E = 320000
D_IN = 128
H = 256
L = 5
NODES_PER_GRAPH = 200
G = N // NODES_PER_GRAPH


def _conv1d(x, w, b):
    y = jax.lax.conv_general_dilated(x, w, window_strides=(1,), padding='VALID',
                                     dimension_numbers=('NCH', 'OIH', 'NCH'))
    return y + b[None, :, None]


def _maxpool1d(x, k, s):
    return jax.lax.reduce_window(x, -jnp.inf, jax.lax.max, (1, 1, k), (1, 1, s), 'VALID')


def _ggc(x, edge_index, ggnn_weight, gru_w_ih, gru_w_hh, gru_b_ih, gru_b_hh):
    h = jnp.pad(x, ((0, 0), (0, H - D_IN)))
    src = edge_index[0]
    dst = edge_index[1]
    for i in range(L):
        m = h @ ggnn_weight[i]
        agg = jnp.zeros((N, H), dtype=x.dtype).at[dst].add(jnp.take(m, src, axis=0))
        gi = agg @ gru_w_ih.T + gru_b_ih
        gh = h @ gru_w_hh.T + gru_b_hh
        i_r, i_z, i_n = jnp.split(gi, 3, axis=1)
        h_r, h_z, h_n = jnp.split(gh, 3, axis=1)
        r = jax.nn.sigmoid(i_r + h_r)
        zt = jax.nn.sigmoid(i_z + h_z)
        n = jnp.tanh(i_n + r * h_n)
        h = (1.0 - zt) * n + zt * h
    return h


def _conv_branch(feat, w1, b1, w2, b2, mw, mb):
    t = feat.reshape(-1, NODES_PER_GRAPH, feat.shape[1])
    t = _conv1d(t, w1, b1)
    t = jax.nn.relu(t)
    t = _maxpool1d(t, 3, 2)
    t = _conv1d(t, w2, b2)
    t = jax.nn.relu(t)
    t = _maxpool1d(t, 2, 2)
    t = t.squeeze(1)
    t = t @ mw.T + mb
    return t.squeeze(1)


def _fuse_kernel(z_ref, y_ref, o_ref):
    o_ref[...] = jax.nn.sigmoid(z_ref[...] * y_ref[...])


def kernel(x, edge_index, batch_index, ggnn_weight, gru_w_ih, gru_w_hh, gru_b_ih, gru_b_hh,
           convw1_w, convw1_b, convw2_w, convw2_b, mlpw_w, mlpw_b,
           convn1_w, convn1_b, convn2_w, convn2_b, mlpn_w, mlpn_b):
    rec = _ggc(x, edge_index, ggnn_weight, gru_w_ih, gru_w_hh, gru_b_ih, gru_b_hh)
    z_in = jnp.concatenate([x, rec], axis=1)
    z = _conv_branch(z_in, convw1_w, convw1_b, convw2_w, convw2_b, mlpw_w, mlpw_b)
    y = _conv_branch(rec, convn1_w, convn1_b, convn2_w, convn2_b, mlpn_w, mlpn_b)
    out = pl.pallas_call(
        _fuse_kernel,
        out_shape=jax.ShapeDtypeStruct((G,), jnp.float32),
    )(z, y)
    return out


# speedup vs baseline: 4.3873x; 4.3873x over previous
import functools

import jax
import jax.numpy as jnp
from jax import lax
from jax.experimental import pallas as pl
from jax.experimental.pallas import tpu as pltpu
from jax.experimental.pallas import tpu_sc as plsc

_N = 10000
_E = 320000
_D_IN = 128
_H = 256
_L = 5
_NPG = 200
_G = _N // _NPG
_K = 128
_NSUB = 16
_CHUNKS = _E // _K
_SL = 624
_SL_LAST = _N - 15 * _SL
_LAST_OFF = 15 * _SL


def _edge_agg_body(m2, src, dst, zeros, out, src_v, dst_v, rows_v, agg_sh, sem):
    c = lax.axis_index("c")
    s = lax.axis_index("s")

    @pl.when(s < 15)
    def _():
        pltpu.sync_copy(zeros.at[pl.ds(0, _SL)], agg_sh.at[pl.ds(s * _SL, _SL)])

    @pl.when(s == 15)
    def _():
        pltpu.sync_copy(zeros, agg_sh.at[pl.ds(_LAST_OFF, _SL_LAST)])

    plsc.subcore_barrier()
    row_off = c * _N

    def chunk_body(j, carry):
        chunk = s + _NSUB * j

        @pl.when(chunk < _CHUNKS)
        def _():
            off = chunk * _K
            pltpu.sync_copy(src.at[pl.ds(off, _K)], src_v)
            pltpu.sync_copy(dst.at[pl.ds(off, _K)], dst_v)
            for i in range(_K // 16):
                sl = pl.ds(i * 16, 16)
                src_v[sl] = src_v[sl] + row_off
            pltpu.async_copy(m2.at[src_v], rows_v, sem).wait()
            pltpu.sync_copy(rows_v, agg_sh.at[dst_v], add=True)

        return carry

    lax.fori_loop(0, (_CHUNKS + _NSUB - 1) // _NSUB, chunk_body, 0)
    plsc.subcore_barrier()

    @pl.when(s < 15)
    def _():
        pltpu.sync_copy(agg_sh.at[pl.ds(s * _SL, _SL)],
                        out.at[pl.ds(row_off + s * _SL, _SL)])

    @pl.when(s == 15)
    def _():
        pltpu.sync_copy(agg_sh.at[pl.ds(_LAST_OFF, _SL_LAST)],
                        out.at[pl.ds(row_off + _LAST_OFF, _SL_LAST)])


def _edge_agg(m2, src, dst, zeros):
    fn = pl.kernel(
        _edge_agg_body,
        mesh=plsc.VectorSubcoreMesh(core_axis_name="c", subcore_axis_name="s"),
        out_type=jax.ShapeDtypeStruct((2 * _N, 128), jnp.float32),
        scratch_types=[
            pltpu.VMEM((_K,), jnp.int32),
            pltpu.VMEM((_K,), jnp.int32),
            pltpu.VMEM((_K, 128), jnp.float32),
            pltpu.VMEM_SHARED((_N, 128), jnp.float32),
            pltpu.SemaphoreType.DMA,
        ],
    )
    return fn(m2, src, dst, zeros)


_R = 2000
_NB = _N // _R


def _m_matmul_body(h_ref, w_ref, o_ref):
    o_ref[...] = jnp.dot(h_ref[...], w_ref[...], preferred_element_type=jnp.float32)


def _m_matmul(h, w):
    return pl.pallas_call(
        _m_matmul_body,
        grid=(2, _NB),
        in_specs=[pl.BlockSpec((_R, _H), lambda hc, i: (i, 0)),
                  pl.BlockSpec((_H, 128), lambda hc, i: (0, hc))],
        out_specs=pl.BlockSpec((_R, 128), lambda hc, i: (hc * _NB + i, 0)),
        out_shape=jax.ShapeDtypeStruct((2 * _N, 128), jnp.float32),
    )(h, w)


def _gru_body(a0_ref, a1_ref, h_ref, wih_ref, whh_ref, bih_ref, bhh_ref, wn_ref,
              hn_ref, mn_ref):
    agg = jnp.concatenate([a0_ref[...], a1_ref[...]], axis=1)
    gi = jnp.dot(agg, wih_ref[...], preferred_element_type=jnp.float32) + bih_ref[...]
    gh = jnp.dot(h_ref[...], whh_ref[...], preferred_element_type=jnp.float32) + bhh_ref[...]
    r = jax.nn.sigmoid(gi[:, :_H] + gh[:, :_H])
    z = jax.nn.sigmoid(gi[:, _H:2 * _H] + gh[:, _H:2 * _H])
    n = jnp.tanh(gi[:, 2 * _H:] + r * gh[:, 2 * _H:])
    hn = (1.0 - z) * n + z * h_ref[...]
    hn_ref[...] = hn
    mn = jnp.dot(hn, wn_ref[...], preferred_element_type=jnp.float32)
    mn_ref[0] = mn[:, :128]
    mn_ref[1] = mn[:, 128:]


def _gru(agg2, h, wih_t, whh_t, bih, bhh, wn):
    hn, mn = pl.pallas_call(
        _gru_body,
        grid=(_NB,),
        in_specs=[
            pl.BlockSpec((_R, 128), lambda i: (i, 0)),
            pl.BlockSpec((_R, 128), lambda i: (_NB + i, 0)),
            pl.BlockSpec((_R, _H), lambda i: (i, 0)),
            pl.BlockSpec((_H, 3 * _H), lambda i: (0, 0)),
            pl.BlockSpec((_H, 3 * _H), lambda i: (0, 0)),
            pl.BlockSpec((1, 3 * _H), lambda i: (0, 0)),
            pl.BlockSpec((1, 3 * _H), lambda i: (0, 0)),
            pl.BlockSpec((_H, _H), lambda i: (0, 0)),
        ],
        out_specs=[pl.BlockSpec((_R, _H), lambda i: (i, 0)),
                   pl.BlockSpec((2, _R, 128), lambda i: (0, i, 0))],
        out_shape=[jax.ShapeDtypeStruct((_N, _H), jnp.float32),
                   jax.ShapeDtypeStruct((2, _N, 128), jnp.float32)],
    )(agg2, agg2, h, wih_t, whh_t, bih, bhh, wn)
    return hn, mn.reshape(2 * _N, 128)


_GB = 5
_HSTEPS = _G // _GB
_WC1 = _D_IN + _H - 2
_NC1 = _H - 2


def _head_body(zin_ref, wall_ref, b1w_ref, b1n_ref, w2w_ref, w2n_ref, mw_ref,
               mn_ref, scal_ref, o_ref):
    b2w = scal_ref[0, 0]
    b2n = scal_ref[0, 1]
    mbw = scal_ref[0, 2]
    mbn = scal_ref[0, 3]
    vals = []
    for g in range(_GB):
        t = zin_ref[g * _NPG:(g + 1) * _NPG, :]
        y = jnp.dot(wall_ref[...], t, preferred_element_type=jnp.float32)
        accw = (y[0:50, 0:_WC1] + y[50:100, 1:_WC1 + 1] + y[100:150, 2:_WC1 + 2])
        r1 = jnp.maximum(accw + b1w_ref[:, 0:1], 0.0)
        s1 = jnp.maximum(jnp.maximum(r1[:, 0:380], r1[:, 1:381]), r1[:, 2:382])
        q = jnp.maximum(
            jnp.dot(w2w_ref[...], s1, preferred_element_type=jnp.float32) + b2w, 0.0)
        t2 = jnp.maximum(q[:, 0:378], q[:, 2:380])
        zg = jnp.sum(t2 * mw_ref[:, 0:378]) + mbw
        accn = (y[150:200, 128:128 + _NC1] + y[200:250, 129:129 + _NC1]
                + y[250:300, 130:130 + _NC1])
        r1n = jnp.maximum(accn + b1n_ref[:, 0:1], 0.0)
        s1n = jnp.maximum(jnp.maximum(r1n[:, 0:252], r1n[:, 1:253]), r1n[:, 2:254])
        qn = jnp.maximum(
            jnp.dot(w2n_ref[...], s1n, preferred_element_type=jnp.float32) + b2n, 0.0)
        t2n = jnp.maximum(qn[:, 0:250], qn[:, 2:252])
        yg = jnp.sum(t2n * mn_ref[:, 0:250]) + mbn
        vals.append(jax.nn.sigmoid(zg * yg))
    o_ref[0, 0, :] = jnp.stack(vals)


def _head(zin, wall, b1w, b1n, w2w, w2n, mw, mn, scal):
    out = pl.pallas_call(
        _head_body,
        grid=(_HSTEPS,),
        in_specs=[
            pl.BlockSpec((_GB * _NPG, _D_IN + _H), lambda i: (i, 0)),
            pl.BlockSpec((300, _NPG), lambda i: (0, 0)),
            pl.BlockSpec((50, 128), lambda i: (0, 0)),
            pl.BlockSpec((50, 128), lambda i: (0, 0)),
            pl.BlockSpec((1, 50), lambda i: (0, 0)),
            pl.BlockSpec((1, 50), lambda i: (0, 0)),
            pl.BlockSpec((1, 380), lambda i: (0, 0)),
            pl.BlockSpec((1, 252), lambda i: (0, 0)),
            pl.BlockSpec((1, 4), lambda i: (0, 0)),
        ],
        out_specs=pl.BlockSpec((1, 1, _GB), lambda i: (i, 0, 0)),
        out_shape=jax.ShapeDtypeStruct((_HSTEPS, 1, _GB), jnp.float32),
    )(zin, wall, b1w, b1n, w2w, w2n, mw, mn, scal)
    return out.reshape(_G)


def kernel(x, edge_index, batch_index, ggnn_weight, gru_w_ih, gru_w_hh, gru_b_ih,
           gru_b_hh, convw1_w, convw1_b, convw2_w, convw2_b, mlpw_w, mlpw_b,
           convn1_w, convn1_b, convn2_w, convn2_b, mlpn_w, mlpn_b):
    src = edge_index[0]
    dst = edge_index[1]
    zeros = jnp.zeros((_SL_LAST, 128), jnp.float32)

    wih_t = gru_w_ih.T
    whh_t = gru_w_hh.T
    bih = gru_b_ih.reshape(1, 3 * _H)
    bhh = gru_b_hh.reshape(1, 3 * _H)

    h = jnp.pad(x, ((0, 0), (0, _H - _D_IN)))
    m2 = _m_matmul(h, ggnn_weight[0])
    for i in range(_L):
        agg2 = _edge_agg(m2, src, dst, zeros)
        h, m2 = _gru(agg2, h, wih_t, whh_t, bih, bhh,
                     ggnn_weight[(i + 1) % _L])

    zin = jnp.concatenate([x, h], axis=1)
    wall = jnp.concatenate(
        [convw1_w[:, :, 0], convw1_w[:, :, 1], convw1_w[:, :, 2],
         convn1_w[:, :, 0], convn1_w[:, :, 1], convn1_w[:, :, 2]], axis=0)
    b1w = jnp.broadcast_to(convw1_b[:, None], (50, 128))
    b1n = jnp.broadcast_to(convn1_b[:, None], (50, 128))
    w2w = convw2_w[:, :, 0]
    w2n = convn2_w[:, :, 0]
    mw = jnp.zeros((1, 380), jnp.float32).at[0, 0:380:4].set(mlpw_w[0])
    mn = jnp.zeros((1, 252), jnp.float32).at[0, 0:252:4].set(mlpn_w[0])
    scal = jnp.stack([convw2_b[0], convn2_b[0], mlpw_b[0], mlpn_b[0]]).reshape(1, 4)
    return _head(zin, wall, b1w, b1n, w2w, w2n, mw, mn, scal)


# speedup vs baseline: 8.9445x; 2.0387x over previous
import functools

import jax
import jax.numpy as jnp
from jax import lax
from jax.experimental import pallas as pl
from jax.experimental.pallas import tpu as pltpu
from jax.experimental.pallas import tpu_sc as plsc

_N = 10000
_E = 320000
_D_IN = 128
_H = 256
_L = 5
_NPG = 200
_G = _N // _NPG
_K = 128
_NSUB = 16
_K = 125
_TCH = (_E // _NSUB) // _K
_GC = 16
_NGROUP = _TCH // _GC
_SL = 624
_SL_LAST = _N - 15 * _SL
_LAST_OFF = 15 * _SL


def _edge_agg_body(m2, srcs, dst, zeros, out, src_g0, src_g1, dst_g0, dst_g1,
                   rows0, rows1, agg_sh, gsem0, gsem1, isem0, isem1):
    c = lax.axis_index("c")
    s = lax.axis_index("s")

    @pl.when(s < 15)
    def _():
        pltpu.sync_copy(zeros.at[pl.ds(0, _SL)], agg_sh.at[pl.ds(s * _SL, _SL)])

    @pl.when(s == 15)
    def _():
        pltpu.sync_copy(zeros, agg_sh.at[pl.ds(_LAST_OFF, _SL_LAST)])

    src_g = (src_g0, src_g1)
    dst_g = (dst_g0, dst_g1)
    isem = (isem0, isem1)

    def idx_copies(g, b):
        off = s * _TCH + g * _GC
        return (pltpu.make_async_copy(srcs.at[c, pl.ds(off, _GC)], src_g[b], isem[b]),
                pltpu.make_async_copy(dst.at[pl.ds(off, _GC)], dst_g[b], isem[b]))

    for cp in idx_copies(0, 0):
        cp.start()
    for cp in idx_copies(1, 1):
        cp.start()

    def gather(sg, jj, rows, sem):
        return pltpu.make_async_copy(m2.at[sg.at[jj]], rows, sem)

    def group_body(u, carry):
        for half in (0, 1):
            g = 2 * u + half
            for cp in idx_copies(g, half):
                cp.wait()
            sg, dg = src_g[half], dst_g[half]
            gather(sg, 0, rows0, gsem0).start()
            gather(sg, 1, rows1, gsem1).start()

            def chunk_body(k, cc, sg=sg, dg=dg):
                j0 = 2 * k
                gather(sg, j0, rows0, gsem0).wait()
                pltpu.sync_copy(rows0, agg_sh.at[dg.at[j0]], add=True)

                @pl.when(k < _GC // 2 - 1)
                def _():
                    gather(sg, j0 + 2, rows0, gsem0).start()

                gather(sg, j0 + 1, rows1, gsem1).wait()
                pltpu.sync_copy(rows1, agg_sh.at[dg.at[j0 + 1]], add=True)

                @pl.when(k < _GC // 2 - 1)
                def _():
                    gather(sg, j0 + 3, rows1, gsem1).start()

                return cc

            lax.fori_loop(0, _GC // 2, chunk_body, 0)

            @pl.when(g + 2 < _NGROUP)
            def _(g=g, half=half):
                for cp in idx_copies(g + 2, half):
                    cp.start()

        return carry

    lax.fori_loop(0, _NGROUP // 2, group_body, 0)
    plsc.subcore_barrier()
    row_off = c * _N

    @pl.when(s < 15)
    def _():
        pltpu.sync_copy(agg_sh.at[pl.ds(s * _SL, _SL)],
                        out.at[pl.ds(row_off + s * _SL, _SL)])

    @pl.when(s == 15)
    def _():
        pltpu.sync_copy(agg_sh.at[pl.ds(_LAST_OFF, _SL_LAST)],
                        out.at[pl.ds(row_off + _LAST_OFF, _SL_LAST)])


def _edge_agg(m2, srcs, dst, zeros):
    fn = pl.kernel(
        _edge_agg_body,
        mesh=plsc.VectorSubcoreMesh(core_axis_name="c", subcore_axis_name="s"),
        out_type=jax.ShapeDtypeStruct((2 * _N, 128), jnp.float32),
        scratch_types=[
            pltpu.VMEM((_GC, _K), jnp.int32),
            pltpu.VMEM((_GC, _K), jnp.int32),
            pltpu.VMEM((_GC, _K), jnp.int32),
            pltpu.VMEM((_GC, _K), jnp.int32),
            pltpu.VMEM((_K, 128), jnp.float32),
            pltpu.VMEM((_K, 128), jnp.float32),
            pltpu.VMEM_SHARED((_N, 128), jnp.float32),
            pltpu.SemaphoreType.DMA,
            pltpu.SemaphoreType.DMA,
            pltpu.SemaphoreType.DMA,
            pltpu.SemaphoreType.DMA,
        ],
    )
    return fn(m2, srcs, dst, zeros)


_R = 2000
_NB = _N // _R


def _m_matmul_body(h_ref, w_ref, o_ref):
    o_ref[...] = jnp.dot(h_ref[...], w_ref[...], preferred_element_type=jnp.float32)


def _m_matmul(h, w):
    return pl.pallas_call(
        _m_matmul_body,
        grid=(2, _NB),
        in_specs=[pl.BlockSpec((_R, _H), lambda hc, i: (i, 0)),
                  pl.BlockSpec((_H, 128), lambda hc, i: (0, hc))],
        out_specs=pl.BlockSpec((_R, 128), lambda hc, i: (hc * _NB + i, 0)),
        out_shape=jax.ShapeDtypeStruct((2 * _N, 128), jnp.float32),
    )(h, w)


def _gru_body(a0_ref, a1_ref, h_ref, wih_ref, whh_ref, bih_ref, bhh_ref, wn_ref,
              hn_ref, mn_ref):
    agg = jnp.concatenate([a0_ref[...], a1_ref[...]], axis=1)
    gi = jnp.dot(agg, wih_ref[...], preferred_element_type=jnp.float32) + bih_ref[...]
    gh = jnp.dot(h_ref[...], whh_ref[...], preferred_element_type=jnp.float32) + bhh_ref[...]
    r = jax.nn.sigmoid(gi[:, :_H] + gh[:, :_H])
    z = jax.nn.sigmoid(gi[:, _H:2 * _H] + gh[:, _H:2 * _H])
    n = jnp.tanh(gi[:, 2 * _H:] + r * gh[:, 2 * _H:])
    hn = (1.0 - z) * n + z * h_ref[...]
    hn_ref[...] = hn
    mn = jnp.dot(hn, wn_ref[...], preferred_element_type=jnp.float32)
    mn_ref[0] = mn[:, :128]
    mn_ref[1] = mn[:, 128:]


def _gru(agg2, h, wih_t, whh_t, bih, bhh, wn):
    hn, mn = pl.pallas_call(
        _gru_body,
        grid=(_NB,),
        in_specs=[
            pl.BlockSpec((_R, 128), lambda i: (i, 0)),
            pl.BlockSpec((_R, 128), lambda i: (_NB + i, 0)),
            pl.BlockSpec((_R, _H), lambda i: (i, 0)),
            pl.BlockSpec((_H, 3 * _H), lambda i: (0, 0)),
            pl.BlockSpec((_H, 3 * _H), lambda i: (0, 0)),
            pl.BlockSpec((1, 3 * _H), lambda i: (0, 0)),
            pl.BlockSpec((1, 3 * _H), lambda i: (0, 0)),
            pl.BlockSpec((_H, _H), lambda i: (0, 0)),
        ],
        out_specs=[pl.BlockSpec((_R, _H), lambda i: (i, 0)),
                   pl.BlockSpec((2, _R, 128), lambda i: (0, i, 0))],
        out_shape=[jax.ShapeDtypeStruct((_N, _H), jnp.float32),
                   jax.ShapeDtypeStruct((2, _N, 128), jnp.float32)],
    )(agg2, agg2, h, wih_t, whh_t, bih, bhh, wn)
    return hn, mn.reshape(2 * _N, 128)


_GB = 5
_HSTEPS = _G // _GB
_WC1 = _D_IN + _H - 2
_NC1 = _H - 2


def _head_body(zin_ref, wall_ref, b1w_ref, b1n_ref, w2w_ref, w2n_ref, mw_ref,
               mn_ref, scal_ref, o_ref):
    b2w = scal_ref[0, 0]
    b2n = scal_ref[0, 1]
    mbw = scal_ref[0, 2]
    mbn = scal_ref[0, 3]
    vals = []
    for g in range(_GB):
        t = zin_ref[g * _NPG:(g + 1) * _NPG, :]
        y = jnp.dot(wall_ref[...], t, preferred_element_type=jnp.float32)
        accw = (y[0:50, 0:_WC1] + y[50:100, 1:_WC1 + 1] + y[100:150, 2:_WC1 + 2])
        r1 = jnp.maximum(accw + b1w_ref[:, 0:1], 0.0)
        s1 = jnp.maximum(jnp.maximum(r1[:, 0:380], r1[:, 1:381]), r1[:, 2:382])
        q = jnp.maximum(
            jnp.dot(w2w_ref[...], s1, preferred_element_type=jnp.float32) + b2w, 0.0)
        t2 = jnp.maximum(q[:, 0:378], q[:, 2:380])
        zg = jnp.sum(t2 * mw_ref[:, 0:378]) + mbw
        accn = (y[150:200, 128:128 + _NC1] + y[200:250, 129:129 + _NC1]
                + y[250:300, 130:130 + _NC1])
        r1n = jnp.maximum(accn + b1n_ref[:, 0:1], 0.0)
        s1n = jnp.maximum(jnp.maximum(r1n[:, 0:252], r1n[:, 1:253]), r1n[:, 2:254])
        qn = jnp.maximum(
            jnp.dot(w2n_ref[...], s1n, preferred_element_type=jnp.float32) + b2n, 0.0)
        t2n = jnp.maximum(qn[:, 0:250], qn[:, 2:252])
        yg = jnp.sum(t2n * mn_ref[:, 0:250]) + mbn
        vals.append(jax.nn.sigmoid(zg * yg))
    o_ref[0, 0, :] = jnp.stack(vals)


def _head(zin, wall, b1w, b1n, w2w, w2n, mw, mn, scal):
    out = pl.pallas_call(
        _head_body,
        grid=(_HSTEPS,),
        in_specs=[
            pl.BlockSpec((_GB * _NPG, _D_IN + _H), lambda i: (i, 0)),
            pl.BlockSpec((300, _NPG), lambda i: (0, 0)),
            pl.BlockSpec((50, 128), lambda i: (0, 0)),
            pl.BlockSpec((50, 128), lambda i: (0, 0)),
            pl.BlockSpec((1, 50), lambda i: (0, 0)),
            pl.BlockSpec((1, 50), lambda i: (0, 0)),
            pl.BlockSpec((1, 380), lambda i: (0, 0)),
            pl.BlockSpec((1, 252), lambda i: (0, 0)),
            pl.BlockSpec((1, 4), lambda i: (0, 0)),
        ],
        out_specs=pl.BlockSpec((1, 1, _GB), lambda i: (i, 0, 0)),
        out_shape=jax.ShapeDtypeStruct((_HSTEPS, 1, _GB), jnp.float32),
    )(zin, wall, b1w, b1n, w2w, w2n, mw, mn, scal)
    return out.reshape(_G)


def kernel(x, edge_index, batch_index, ggnn_weight, gru_w_ih, gru_w_hh, gru_b_ih,
           gru_b_hh, convw1_w, convw1_b, convw2_w, convw2_b, mlpw_w, mlpw_b,
           convn1_w, convn1_b, convn2_w, convn2_b, mlpn_w, mlpn_b):
    src = edge_index[0]
    srcs = jnp.stack([src, src + _N]).reshape(2, _E // _K, _K)
    dst = edge_index[1].reshape(_E // _K, _K)
    zeros = jnp.zeros((_SL_LAST, 128), jnp.float32)

    wih_t = gru_w_ih.T
    whh_t = gru_w_hh.T
    bih = gru_b_ih.reshape(1, 3 * _H)
    bhh = gru_b_hh.reshape(1, 3 * _H)

    h = jnp.pad(x, ((0, 0), (0, _H - _D_IN)))
    m2 = _m_matmul(h, ggnn_weight[0])
    for i in range(_L):
        agg2 = _edge_agg(m2, srcs, dst, zeros)
        h, m2 = _gru(agg2, h, wih_t, whh_t, bih, bhh,
                     ggnn_weight[(i + 1) % _L])

    zin = jnp.concatenate([x, h], axis=1)
    wall = jnp.concatenate(
        [convw1_w[:, :, 0], convw1_w[:, :, 1], convw1_w[:, :, 2],
         convn1_w[:, :, 0], convn1_w[:, :, 1], convn1_w[:, :, 2]], axis=0)
    b1w = jnp.broadcast_to(convw1_b[:, None], (50, 128))
    b1n = jnp.broadcast_to(convn1_b[:, None], (50, 128))
    w2w = convw2_w[:, :, 0]
    w2n = convn2_w[:, :, 0]
    mw = jnp.zeros((1, 380), jnp.float32).at[0, 0:380:4].set(mlpw_w[0])
    mn = jnp.zeros((1, 252), jnp.float32).at[0, 0:252:4].set(mlpn_w[0])
    scal = jnp.stack([convw2_b[0], convn2_b[0], mlpw_b[0], mlpn_b[0]]).reshape(1, 4)
    return _head(zin, wall, b1w, b1n, w2w, w2n, mw, mn, scal)
